# baseline (speedup 1.0000x reference)
import jax
import jax.numpy as jnp
import numpy as np
from jax.experimental import pallas as pl

N = 32768
D = 64
FF = 48
H = 3 * FF

BN = 4096


def _moe_block(x_ref, w1a_ref, b1a_ref, w1b_ref, b1b_ref, gw_ref, gb_ref,
               ind_ref, w2_ref, out_ref):
    x = x_ref[...]

    ha = jnp.dot(x, w1a_ref[...], preferred_element_type=jnp.float32) + b1a_ref[...]
    hb = jnp.dot(x, w1b_ref[...], preferred_element_type=jnp.float32) + b1b_ref[...]
    act = (ha * jax.nn.sigmoid(ha)) * hb

    lg = jnp.dot(x, gw_ref[...], preferred_element_type=jnp.float32) + gb_ref[...]
    l0 = lg[:, 0:1]
    l1 = lg[:, 1:2]
    pick1 = (l1 > l0).astype(jnp.float32)
    w = jax.nn.sigmoid(jnp.abs(l1 - l0))
    s1 = w * pick1
    s0 = w - s1
    sv = jnp.concatenate([jnp.ones_like(w), s0, s1], axis=1)

    scale = jnp.dot(sv, ind_ref[...], preferred_element_type=jnp.float32)
    act2 = jnp.concatenate([act * scale, sv], axis=1)

    out_ref[...] = jnp.dot(act2, w2_ref[...], preferred_element_type=jnp.float32)


@jax.jit
def kernel(x, sw1, sb1, sw2, sb2, rw1, rb1, rw2, rb2, gw, gb):
    w1a = jnp.concatenate([sw1[:, :FF], rw1[0][:, :FF], rw1[1][:, :FF]], axis=1)
    w1b = jnp.concatenate([sw1[:, FF:], rw1[0][:, FF:], rw1[1][:, FF:]], axis=1)
    b1a = jnp.concatenate([sb1[:FF], rb1[0][:FF], rb1[1][:FF]], axis=0)[None]
    b1b = jnp.concatenate([sb1[FF:], rb1[0][FF:], rb1[1][FF:]], axis=0)[None]

    ind = np.zeros((3, H), np.float32)
    ind[0, :FF] = 1.0
    ind[1, FF:2 * FF] = 1.0
    ind[2, 2 * FF:] = 1.0
    ind = jnp.asarray(ind)

    w2 = jnp.concatenate(
        [sw2, rw2[0], rw2[1], sb2[None, :], rb2[0][None, :], rb2[1][None, :]],
        axis=0)

    grid = (N // BN,)
    full = lambda *s: pl.BlockSpec(s, lambda i: (0,) * len(s))
    return pl.pallas_call(
        _moe_block,
        grid=grid,
        in_specs=[
            pl.BlockSpec((BN, D), lambda i: (i, 0)),
            full(D, H), full(1, H), full(D, H), full(1, H),
            full(D, 2), full(2), full(3, H), full(H + 3, D),
        ],
        out_specs=pl.BlockSpec((BN, D), lambda i: (i, 0)),
        out_shape=jax.ShapeDtypeStruct((N, D), jnp.float32),
    )(x, w1a, b1a, w1b, b1b, gw, gb, ind, w2)


# speedup vs baseline: 1.1464x; 1.1464x over previous
import jax
import jax.numpy as jnp
from jax.experimental import pallas as pl

N = 32768
D = 64
FF = 48

BN = 4096


def _moe_block(x_ref, sw1_ref, sb1_ref, sw2_ref, sb2_ref,
               rw1_ref, rb1_ref, rw2_ref, rb2_ref, gw_ref, gb_ref,
               out_ref):
    x = x_ref[...]

    def expert(w1, b1, w2, b2):
        h = jnp.dot(x, w1, preferred_element_type=jnp.float32) + b1
        a = h[:, :FF]
        b = h[:, FF:]
        act = (a * jax.nn.sigmoid(a)) * b
        return jnp.dot(act, w2, preferred_element_type=jnp.float32) + b2

    shared = expert(sw1_ref[...], sb1_ref[...], sw2_ref[...], sb2_ref[...])
    o0 = expert(rw1_ref[0], rb1_ref[0], rw2_ref[0], rb2_ref[0])
    o1 = expert(rw1_ref[1], rb1_ref[1], rw2_ref[1], rb2_ref[1])

    logits = jnp.dot(x, gw_ref[...], preferred_element_type=jnp.float32) + gb_ref[...]
    l0 = logits[:, 0:1]
    l1 = logits[:, 1:2]
    pick1 = l1 > l0
    w = jax.nn.sigmoid(jnp.abs(l1 - l0))
    routed = jnp.where(pick1, o1, o0) * w
    out_ref[...] = shared + routed


@jax.jit
def kernel(x, sw1, sb1, sw2, sb2, rw1, rb1, rw2, rb2, gw, gb):
    grid = (N // BN,)
    full = lambda *s: pl.BlockSpec(s, lambda i: (0,) * len(s))
    return pl.pallas_call(
        _moe_block,
        grid=grid,
        in_specs=[
            pl.BlockSpec((BN, D), lambda i: (i, 0)),
            full(D, 2 * FF), full(2 * FF), full(FF, D), full(D),
            full(2, D, 2 * FF), full(2, 2 * FF), full(2, FF, D), full(2, D),
            full(D, 2), full(2),
        ],
        out_specs=pl.BlockSpec((BN, D), lambda i: (i, 0)),
        out_shape=jax.ShapeDtypeStruct((N, D), jnp.float32),
    )(x, sw1, sb1, sw2, sb2, rw1, rb1, rw2, rb2, gw, gb)
